# baseline (speedup 1.0000x reference)
import functools

import jax
import jax.numpy as jnp
from jax.experimental import pallas as pl
from jax.experimental.pallas import tpu as pltpu

INPUT_DIM = 512
NUM_EMB = 8192
EMB_DIM = 64
NUM_SPLITS = 8
BATCH = 4096
LAT = EMB_DIM * NUM_SPLITS

BT = 256


def _vq_kernel(x_ref, We_ref, be_ref, cbs_ref, csT_ref, Wd_ref, bd_ref,
               xr_ref, q_ref, codes_ref, perp_ref, counts_ref,
               z_scr, xr_scr, codes_scr, sE_scr):
    s = pl.program_id(0)
    b = pl.program_id(1)
    nb = pl.num_programs(1)
    row0 = b * BT

    @pl.when((s == 0) & (b == 0))
    def _init():
        counts_ref[...] = jnp.zeros_like(counts_ref)

    @pl.when(s == 0)
    def _encode():
        z = jnp.dot(x_ref[...], We_ref[...]) + be_ref[...]
        for ss in range(NUM_SPLITS):
            z_scr[ss, pl.ds(row0, BT), :] = z[:, ss * EMB_DIM:(ss + 1) * EMB_DIM]

    cs = cbs_ref[0]
    csT = csT_ref[0]

    @pl.when(b == 0)
    def _se():
        ones = jnp.ones((1, EMB_DIM), jnp.float32)
        sE_scr[...] = jnp.dot(ones, csT * csT,
                              precision=jax.lax.Precision.HIGHEST) * 0.25

    flat = z_scr[s, pl.ds(row0, BT), :]
    m2 = jnp.dot(flat, csT)
    s_flat = jnp.sum(flat * flat, axis=1, keepdims=True)
    d = (s_flat + sE_scr[...]) + m2
    dmin = jnp.min(d, axis=1, keepdims=True)
    iota = jax.lax.broadcasted_iota(
        jnp.int32, (BT, NUM_EMB), 1).astype(jnp.float32)
    key = jnp.where(d == dmin, iota, float(NUM_EMB))
    idxf = jnp.min(key, axis=1, keepdims=True)
    idx = idxf.astype(jnp.int32)
    onehot = jnp.where(key == idxf, 1.0, 0.0)
    qs = jnp.dot(onehot, cs) * -0.5
    q_ref[...] = qs.reshape(1, BT, EMB_DIM)

    cgrp = jax.lax.broadcasted_iota(jnp.int32, (BT, NUM_SPLITS), 1)
    old_c = codes_scr[pl.ds(row0, BT), :]
    codes_scr[pl.ds(row0, BT), :] = jnp.where(cgrp == s, idx, old_c)

    colsum = jnp.sum(onehot, axis=0).reshape(1, NUM_EMB)
    rgrp = jax.lax.broadcasted_iota(jnp.int32, (NUM_SPLITS, NUM_EMB), 0)
    counts_ref[...] = counts_ref[...] + jnp.where(rgrp == s, colsum, 0.0)

    part = jnp.dot(qs, Wd_ref[0])

    @pl.when(s == 0)
    def _dec0():
        xr_scr[pl.ds(row0, BT), :] = part

    @pl.when(s > 0)
    def _dec():
        xr_scr[pl.ds(row0, BT), :] = xr_scr[pl.ds(row0, BT), :] + part

    @pl.when(s == NUM_SPLITS - 1)
    def _emit():
        xr_ref[...] = xr_scr[pl.ds(row0, BT), :] + bd_ref[...]
        codes_ref[...] = codes_scr[pl.ds(row0, BT), :]

    @pl.when((s == NUM_SPLITS - 1) & (b == nb - 1))
    def _finish():
        avg = counts_ref[...] * (1.0 / BATCH)
        plogp = avg * jnp.log(avg + 1e-10)
        ent = jnp.sum(plogp, axis=1, keepdims=True)
        perps = jnp.exp(-ent)
        val = jnp.sum(perps) * (1.0 / NUM_SPLITS)
        perp_ref[...] = jnp.full((1, 128), val, dtype=jnp.float32)


@functools.partial(jax.jit, static_argnames=())
def kernel(x, W_enc, b_enc, codebooks, W_dec, b_dec):
    nb = BATCH // BT
    be2 = b_enc.reshape(1, LAT)
    bd2 = b_dec.reshape(1, INPUT_DIM)
    cbs = codebooks * (-2.0)
    cbsT = jnp.transpose(cbs, (0, 2, 1))
    Wd3 = W_dec.reshape(NUM_SPLITS, EMB_DIM, INPUT_DIM)
    out_shapes = (
        jax.ShapeDtypeStruct((BATCH, INPUT_DIM), jnp.float32),
        jax.ShapeDtypeStruct((NUM_SPLITS, BATCH, EMB_DIM), jnp.float32),
        jax.ShapeDtypeStruct((BATCH, NUM_SPLITS), jnp.int32),
        jax.ShapeDtypeStruct((1, 128), jnp.float32),
        jax.ShapeDtypeStruct((NUM_SPLITS, NUM_EMB), jnp.float32),
    )
    grid_spec = pltpu.PrefetchScalarGridSpec(
        num_scalar_prefetch=0,
        grid=(NUM_SPLITS, nb),
        scratch_shapes=[
            pltpu.VMEM((NUM_SPLITS, BATCH, EMB_DIM), jnp.float32),
            pltpu.VMEM((BATCH, INPUT_DIM), jnp.float32),
            pltpu.VMEM((BATCH, NUM_SPLITS), jnp.int32),
            pltpu.VMEM((1, NUM_EMB), jnp.float32),
        ],
        in_specs=[
            pl.BlockSpec((BT, INPUT_DIM), lambda s, b: (b, 0)),
            pl.BlockSpec((INPUT_DIM, LAT), lambda s, b: (0, 0)),
            pl.BlockSpec((1, LAT), lambda s, b: (0, 0)),
            pl.BlockSpec((1, NUM_EMB, EMB_DIM), lambda s, b: (s, 0, 0)),
            pl.BlockSpec((1, EMB_DIM, NUM_EMB), lambda s, b: (s, 0, 0)),
            pl.BlockSpec((1, EMB_DIM, INPUT_DIM), lambda s, b: (s, 0, 0)),
            pl.BlockSpec((1, INPUT_DIM), lambda s, b: (0, 0)),
        ],
        out_specs=(
            pl.BlockSpec((BT, INPUT_DIM), lambda s, b: (b, 0)),
            pl.BlockSpec((1, BT, EMB_DIM), lambda s, b: (s, b, 0)),
            pl.BlockSpec((BT, NUM_SPLITS), lambda s, b: (b, 0)),
            pl.BlockSpec((1, 128), lambda s, b: (0, 0)),
            pl.BlockSpec((NUM_SPLITS, NUM_EMB), lambda s, b: (0, 0)),
        ),
    )
    x_recon, q_sm, codes, perp, _counts = pl.pallas_call(
        _vq_kernel,
        grid_spec=grid_spec,
        out_shape=out_shapes,
    )(x, W_enc, be2, cbs, cbsT, Wd3, bd2)
    quantized = q_sm.transpose(1, 0, 2).reshape(BATCH, LAT)
    return x_recon, quantized, codes, perp[0, 0]


# speedup vs baseline: 1.6264x; 1.6264x over previous
import dataclasses
import functools

import jax
import jax.numpy as jnp
from jax.experimental import pallas as pl
from jax.experimental.pallas import tpu as pltpu
from jax.experimental.pallas import tpu_sc as plsc

INPUT_DIM = 512
NUM_EMB = 8192
EMB_DIM = 64
NUM_SPLITS = 8
BATCH = 4096
LAT = EMB_DIM * NUM_SPLITS
NBINS = NUM_SPLITS * NUM_EMB

BT = 128
KCH = 512
NCH = NUM_EMB // KCH

SC_CORES = 2
SC_SUBCORES = 16
SC_UNITS = SC_CORES * SC_SUBCORES
GW = 128
N_IDX = BATCH * NUM_SPLITS


def _argmin_kernel(x_ref, We_ref, be_ref, csT_ref,
                   codes_ref, fcodes_ref,
                   z_scr, codes_scr, sE_scr):
    s = pl.program_id(0)
    b = pl.program_id(1)
    row0 = b * BT

    @pl.when(s == 0)
    def _encode():
        z = jnp.dot(x_ref[...], We_ref[...]) + be_ref[...]
        for ss in range(NUM_SPLITS):
            z_scr[ss, pl.ds(row0, BT), :] = z[:, ss * EMB_DIM:(ss + 1) * EMB_DIM]

    csT = csT_ref[0]

    @pl.when(b == 0)
    def _se():
        ones = jnp.ones((1, EMB_DIM), jnp.float32)
        sE_scr[...] = jnp.dot(ones, csT * csT,
                              precision=jax.lax.Precision.HIGHEST) * 0.25

    flat = z_scr[s, pl.ds(row0, BT), :]
    m2 = jnp.dot(flat, csT)
    s_flat = jnp.sum(flat * flat, axis=1, keepdims=True)

    acc = None
    for c in range(NCH):
        dc = (s_flat + sE_scr[0:1, c * KCH:(c + 1) * KCH]) \
            + m2[:, c * KCH:(c + 1) * KCH]
        acc = dc if acc is None else jnp.minimum(acc, dc)
    dmin = jnp.min(acc, axis=1, keepdims=True)

    kacc = None
    for c in range(NCH):
        dc = (s_flat + sE_scr[0:1, c * KCH:(c + 1) * KCH]) \
            + m2[:, c * KCH:(c + 1) * KCH]
        io = (jax.lax.broadcasted_iota(jnp.int32, (BT, KCH), 1)
              + c * KCH).astype(jnp.float32)
        kc = jnp.where(dc == dmin, io, float(NUM_EMB))
        kacc = kc if kacc is None else jnp.minimum(kacc, kc)
    idx = jnp.min(kacc, axis=1, keepdims=True).astype(jnp.int32)

    cgrp = jax.lax.broadcasted_iota(jnp.int32, (BT, NUM_SPLITS), 1)
    old_c = codes_scr[pl.ds(row0, BT), :]
    codes_scr[pl.ds(row0, BT), :] = jnp.where(cgrp == s, idx, old_c)

    @pl.when(s == NUM_SPLITS - 1)
    def _emit():
        c = codes_scr[pl.ds(row0, BT), :]
        codes_ref[...] = c
        fcodes_ref[...] = c + cgrp * NUM_EMB


def _run_argmin(x, W_enc, be2, cbsT):
    nb = BATCH // BT
    grid_spec = pltpu.PrefetchScalarGridSpec(
        num_scalar_prefetch=0,
        grid=(NUM_SPLITS, nb),
        scratch_shapes=[
            pltpu.VMEM((NUM_SPLITS, BATCH, EMB_DIM), jnp.float32),
            pltpu.VMEM((BATCH, NUM_SPLITS), jnp.int32),
            pltpu.VMEM((1, NUM_EMB), jnp.float32),
        ],
        in_specs=[
            pl.BlockSpec((BT, INPUT_DIM), lambda s, b: (b, 0)),
            pl.BlockSpec((INPUT_DIM, LAT), lambda s, b: (0, 0)),
            pl.BlockSpec((1, LAT), lambda s, b: (0, 0)),
            pl.BlockSpec((1, EMB_DIM, NUM_EMB), lambda s, b: (s, 0, 0)),
        ],
        out_specs=(
            pl.BlockSpec((BT, NUM_SPLITS), lambda s, b: (b, 0)),
            pl.BlockSpec((BT, NUM_SPLITS), lambda s, b: (b, 0)),
        ),
    )
    return pl.pallas_call(
        _argmin_kernel,
        grid_spec=grid_spec,
        out_shape=(
            jax.ShapeDtypeStruct((BATCH, NUM_SPLITS), jnp.int32),
            jax.ShapeDtypeStruct((BATCH, NUM_SPLITS), jnp.int32),
        ),
    )(x, W_enc, be2, cbsT)


def _run_sc_gather(cb_flat, fcodes_flat):
    vector_mesh = plsc.VectorSubcoreMesh(
        core_axis_name="core", subcore_axis_name="subcore")
    cp = pltpu.CompilerParams()
    if "needs_layout_passes" in pltpu.CompilerParams.__dataclass_fields__:
        cp = dataclasses.replace(cp, needs_layout_passes=False)

    @pl.kernel(
        out_type=(
            jax.ShapeDtypeStruct((N_IDX, 128), jnp.float32),
            jax.ShapeDtypeStruct((SC_UNITS, NBINS), jnp.float32),
        ),
        mesh=vector_mesh,
        scratch_types=[pltpu.VMEM((NBINS,), jnp.float32)],
        compiler_params=cp,
    )
    def sc_kernel(cb_hbm, i_hbm, q_hbm, h_hbm, hist_ref):
        @pl.loop(0, NBINS, step=16)
        def _zero(i):
            hist_ref[pl.ds(i, 16)] = jnp.zeros((16,), jnp.float32)

        def body(i_vmem, o_vmem):
            pltpu.sync_copy(cb_hbm.at[i_vmem.at[0]], o_vmem)
            for j in range(GW // 16):
                iv = i_vmem[0, pl.ds(j * 16, 16)]
                plsc.addupdate_scatter(hist_ref, [iv],
                                       jnp.ones((16,), jnp.float32))

        pltpu.emit_pipeline(
            body,
            grid=(N_IDX // GW,),
            in_specs=[pl.BlockSpec((1, GW), index_map=lambda i: (0, i))],
            out_specs=[pl.BlockSpec((GW, 128), index_map=lambda i: (i, 0))],
            core_axis_name=("core", "subcore"),
            dimension_semantics=(pltpu.PARALLEL,),
        )(i_hbm, q_hbm)

        gidx = (jax.lax.axis_index("core") * SC_SUBCORES
                + jax.lax.axis_index("subcore"))
        pltpu.sync_copy(hist_ref, h_hbm.at[gidx])

    return sc_kernel(cb_flat, fcodes_flat)


DT = 256


def _decoder_kernel(q_ref, Wd_ref, bd_ref, hist_ref, xr_ref, perp_ref):
    t = pl.program_id(0)
    nt = pl.num_programs(0)
    xr_ref[...] = jnp.dot(q_ref[...], Wd_ref[...]) + bd_ref[...]

    @pl.when(t == nt - 1)
    def _perp():
        h = hist_ref[...]
        counts = jnp.sum(h, axis=0)
        avg = counts * (1.0 / BATCH)
        plogp = avg * jnp.log(avg + 1e-10)
        ent = jnp.sum(plogp, axis=1, keepdims=True)
        perps = jnp.exp(-ent)
        val = jnp.sum(perps) * (1.0 / NUM_SPLITS)
        perp_ref[...] = jnp.full((1, 128), val, dtype=jnp.float32)


def _run_decoder(quantized, W_dec, bd2, hist3):
    nt = BATCH // DT
    grid_spec = pltpu.PrefetchScalarGridSpec(
        num_scalar_prefetch=0,
        grid=(nt,),
        scratch_shapes=[],
        in_specs=[
            pl.BlockSpec((DT, LAT), lambda t: (t, 0)),
            pl.BlockSpec((LAT, INPUT_DIM), lambda t: (0, 0)),
            pl.BlockSpec((1, INPUT_DIM), lambda t: (0, 0)),
            pl.BlockSpec((SC_UNITS, NUM_SPLITS, NUM_EMB), lambda t: (0, 0, 0)),
        ],
        out_specs=(
            pl.BlockSpec((DT, INPUT_DIM), lambda t: (t, 0)),
            pl.BlockSpec((1, 128), lambda t: (0, 0)),
        ),
    )
    return pl.pallas_call(
        _decoder_kernel,
        grid_spec=grid_spec,
        out_shape=(
            jax.ShapeDtypeStruct((BATCH, INPUT_DIM), jnp.float32),
            jax.ShapeDtypeStruct((1, 128), jnp.float32),
        ),
    )(quantized, W_dec, bd2, hist3)


@functools.partial(jax.jit, static_argnames=())
def kernel(x, W_enc, b_enc, codebooks, W_dec, b_dec):
    be2 = b_enc.reshape(1, LAT)
    bd2 = b_dec.reshape(1, INPUT_DIM)
    cbsT = jnp.transpose(codebooks * (-2.0), (0, 2, 1))
    cb_flat = codebooks.reshape(NBINS, EMB_DIM)
    cb_pad = jnp.pad(cb_flat, ((0, 0), (0, 128 - EMB_DIM)))

    codes, fcodes = _run_argmin(x, W_enc, be2, cbsT)
    q_rows, hist = _run_sc_gather(cb_pad, fcodes.reshape(1, N_IDX))
    quantized = q_rows[:, :EMB_DIM].reshape(BATCH, LAT)
    hist3 = hist.reshape(SC_UNITS, NUM_SPLITS, NUM_EMB)
    x_recon, perp = _run_decoder(quantized, W_dec, bd2, hist3)
    return x_recon, quantized, codes, perp[0, 0]
